# baseline (speedup 1.0000x reference)
import jax
import jax.numpy as jnp
from jax.experimental import pallas as pl
from jax.experimental.pallas import tpu as pltpu


def _round_up(x, m):
    return ((x + m - 1) // m) * m


def _fast_exp(s):
    log2e = jnp.float32(1.4426950408889634)
    ln2_hi = jnp.float32(0.6931471824645996)
    ln2_lo = jnp.float32(-1.904654323148236e-09)
    kf = jnp.round(s * log2e)
    r = (s - kf * ln2_hi) - kf * ln2_lo
    p = jnp.float32(1.0 / 24.0)
    p = p * r + jnp.float32(1.0 / 6.0)
    p = p * r + jnp.float32(0.5)
    p = p * r + jnp.float32(1.0)
    p = p * r + jnp.float32(1.0)
    k = kf.astype(jnp.int32)
    scale = jax.lax.bitcast_convert_type((k + 127) << 23, jnp.float32)
    return p * scale


def _make_body(n_steps):
    def body(x_ref, wsm_ref, bsm_ref, wpi_ref, bpi_ref,
             pi_ref, sigma_ref, mu_ref,
             pi_s, sigma_s, mu_s, sems):
        i = pl.program_id(0)
        tb = x_ref.shape[0]
        go = sigma_s.shape[-1]

        x = x_ref[...].astype(jnp.bfloat16)
        sm = jnp.dot(x, wsm_ref[...],
                     preferred_element_type=jnp.float32) + bsm_ref[...]
        row = pl.ds(i * tb, tb)
        sigma_s[row, :] = jnp.exp(sm[:, :go])
        mu_s[row, :] = sm[:, go:]

        logits = jnp.dot(x, wpi_ref[...],
                         preferred_element_type=jnp.float32) + bpi_ref[...]
        m = jnp.max(logits, axis=1, keepdims=True)
        e = _fast_exp(logits - m)
        pi_s[row, :] = e / jnp.sum(e, axis=1, keepdims=True)

        @pl.when(i == n_steps - 1)
        def _flush():
            cp_pi = pltpu.make_async_copy(pi_s, pi_ref, sems.at[0])
            cp_sg = pltpu.make_async_copy(sigma_s, sigma_ref, sems.at[1])
            cp_mu = pltpu.make_async_copy(mu_s, mu_ref, sems.at[2])
            cp_pi.start()
            cp_sg.start()
            cp_mu.start()
            cp_pi.wait()
            cp_sg.wait()
            cp_mu.wait()

    return body


def kernel(x, w_pi, b_pi, w_sigma, b_sigma, w_mu, b_mu):
    B, D = x.shape
    G = w_pi.shape[1]
    GO = w_sigma.shape[1]
    O = GO // G
    out_dtype = x.dtype

    w_sm = jnp.concatenate([w_sigma, w_mu], axis=1).astype(jnp.bfloat16)
    b_sm = jnp.concatenate([b_sigma, b_mu], axis=1)
    w_pi16 = w_pi.astype(jnp.bfloat16)

    TB = min(2048, max(8, _round_up(-(-B // 4), 8)))
    B_pad = _round_up(B, TB)
    if B_pad != B:
        x = jnp.pad(x, ((0, B_pad - B), (0, 0)))
    n = B_pad // TB
    grid = (n,)

    pi_pad, sigma_pad, mu_pad = pl.pallas_call(
        _make_body(n),
        out_shape=(
            jax.ShapeDtypeStruct((B_pad, G), out_dtype),
            jax.ShapeDtypeStruct((B_pad, GO), out_dtype),
            jax.ShapeDtypeStruct((B_pad, GO), out_dtype),
        ),
        grid=grid,
        in_specs=[
            pl.BlockSpec((TB, D), lambda i: (i, 0)),
            pl.BlockSpec((D, 2 * GO), lambda i: (0, 0)),
            pl.BlockSpec((1, 2 * GO), lambda i: (0, 0)),
            pl.BlockSpec((D, G), lambda i: (0, 0)),
            pl.BlockSpec((1, G), lambda i: (0, 0)),
        ],
        out_specs=(
            pl.BlockSpec(memory_space=pl.ANY),
            pl.BlockSpec(memory_space=pl.ANY),
            pl.BlockSpec(memory_space=pl.ANY),
        ),
        scratch_shapes=[
            pltpu.VMEM((B_pad, G), jnp.float32),
            pltpu.VMEM((B_pad, GO), jnp.float32),
            pltpu.VMEM((B_pad, GO), jnp.float32),
            pltpu.SemaphoreType.DMA((3,)),
        ],
        compiler_params=pltpu.CompilerParams(
            dimension_semantics=("arbitrary",),
            vmem_limit_bytes=64 * 1024 * 1024,
        ),
    )(x, w_sm, b_sm, w_pi16, b_pi)

    if B_pad != B:
        pi_pad = pi_pad[:B]
        sigma_pad = sigma_pad[:B]
        mu_pad = mu_pad[:B]
    return pi_pad, sigma_pad.reshape(B, G, O), mu_pad.reshape(B, G, O)


# speedup vs baseline: 1.0914x; 1.0914x over previous
import jax
import jax.numpy as jnp
from jax.experimental import pallas as pl
from jax.experimental.pallas import tpu as pltpu


def _round_up(x, m):
    return ((x + m - 1) // m) * m


def _mdn_body(x_ref, wsm_ref, bsm_ref, wpi_ref, bpi_ref,
              pi_ref, sigma_ref, mu_ref):
    tb = pi_ref.shape[0]
    i = pl.program_id(0)
    x = x_ref[pl.ds(i * tb, tb), :].astype(jnp.bfloat16)
    go = sigma_ref.shape[-1]

    sm = jnp.dot(x, wsm_ref[...],
                 preferred_element_type=jnp.float32) + bsm_ref[...]
    sigma_ref[...] = jnp.exp(sm[:, :go]).astype(sigma_ref.dtype)
    mu_ref[...] = sm[:, go:].astype(mu_ref.dtype)

    logits = jnp.dot(x, wpi_ref[...],
                     preferred_element_type=jnp.float32) + bpi_ref[...]
    m = jnp.max(logits, axis=1, keepdims=True)
    e = jnp.exp(logits - m)
    pi_ref[...] = (e / jnp.sum(e, axis=1, keepdims=True)).astype(pi_ref.dtype)


def kernel(x, w_pi, b_pi, w_sigma, b_sigma, w_mu, b_mu):
    B, D = x.shape
    G = w_pi.shape[1]
    GO = w_sigma.shape[1]
    O = GO // G
    out_dtype = x.dtype

    w_sm = jnp.concatenate([w_sigma, w_mu], axis=1).astype(jnp.bfloat16)
    b_sm = jnp.concatenate([b_sigma, b_mu], axis=1)
    w_pi16 = w_pi.astype(jnp.bfloat16)

    TB = min(4096, max(8, _round_up(-(-B // 4), 8)))
    B_pad = _round_up(B, TB)
    if B_pad != B:
        x = jnp.pad(x, ((0, B_pad - B), (0, 0)))
    grid = (B_pad // TB,)

    pi_pad, sigma_pad, mu_pad = pl.pallas_call(
        _mdn_body,
        out_shape=(
            jax.ShapeDtypeStruct((B_pad, G), out_dtype),
            jax.ShapeDtypeStruct((B_pad, GO), out_dtype),
            jax.ShapeDtypeStruct((B_pad, GO), out_dtype),
        ),
        grid=grid,
        in_specs=[
            pl.BlockSpec((B_pad, D), lambda i: (0, 0)),
            pl.BlockSpec((D, 2 * GO), lambda i: (0, 0)),
            pl.BlockSpec((1, 2 * GO), lambda i: (0, 0)),
            pl.BlockSpec((D, G), lambda i: (0, 0)),
            pl.BlockSpec((1, G), lambda i: (0, 0)),
        ],
        out_specs=(
            pl.BlockSpec((TB, G), lambda i: (i, 0)),
            pl.BlockSpec((TB, GO), lambda i: (i, 0)),
            pl.BlockSpec((TB, GO), lambda i: (i, 0)),
        ),
        compiler_params=pltpu.CompilerParams(
            dimension_semantics=("arbitrary",),
            vmem_limit_bytes=64 * 1024 * 1024,
        ),
    )(x, w_sm, b_sm, w_pi16, b_pi)

    if B_pad != B:
        pi_pad = pi_pad[:B]
        sigma_pad = sigma_pad[:B]
        mu_pad = mu_pad[:B]
    return pi_pad, sigma_pad.reshape(B, G, O), mu_pad.reshape(B, G, O)


# speedup vs baseline: 1.1373x; 1.0421x over previous
import jax
import jax.numpy as jnp
from jax.experimental import pallas as pl
from jax.experimental.pallas import tpu as pltpu


def _round_up(x, m):
    return ((x + m - 1) // m) * m


def _mdn_body(x_ref, wsm_ref, bsm_ref, wpi_ref, bpi_ref,
              pi_ref, sigma_ref, mu_ref):
    x = x_ref[...].astype(jnp.bfloat16)
    go = sigma_ref.shape[-1]

    sm = jnp.dot(x, wsm_ref[...],
                 preferred_element_type=jnp.float32) + bsm_ref[...]
    sigma_ref[...] = jnp.exp(sm[:, :go]).astype(sigma_ref.dtype)
    mu_ref[...] = sm[:, go:].astype(mu_ref.dtype)

    logits = jnp.dot(x, wpi_ref[...],
                     preferred_element_type=jnp.float32) + bpi_ref[...]
    m = jnp.max(logits, axis=1, keepdims=True)
    e = jnp.exp(logits - m)
    pi_ref[...] = (e / jnp.sum(e, axis=1, keepdims=True)).astype(pi_ref.dtype)


def kernel(x, w_pi, b_pi, w_sigma, b_sigma, w_mu, b_mu):
    B, D = x.shape
    G = w_pi.shape[1]
    GO = w_sigma.shape[1]
    O = GO // G
    out_dtype = x.dtype

    w_sm = jnp.concatenate([w_sigma, w_mu], axis=1).astype(jnp.bfloat16)
    b_sm = jnp.concatenate([b_sigma, b_mu], axis=1)
    w_pi16 = w_pi.astype(jnp.bfloat16)

    TB = min(4096, max(8, _round_up(-(-B // 4), 8)))
    B_pad = _round_up(B, TB)
    if B_pad != B:
        x = jnp.pad(x, ((0, B_pad - B), (0, 0)))
    grid = (B_pad // TB,)

    pi_pad, sigma_pad, mu_pad = pl.pallas_call(
        _mdn_body,
        out_shape=(
            jax.ShapeDtypeStruct((B_pad, G), out_dtype),
            jax.ShapeDtypeStruct((B_pad, GO), out_dtype),
            jax.ShapeDtypeStruct((B_pad, GO), out_dtype),
        ),
        grid=grid,
        in_specs=[
            pl.BlockSpec((TB, D), lambda i: (i, 0)),
            pl.BlockSpec((D, 2 * GO), lambda i: (0, 0)),
            pl.BlockSpec((1, 2 * GO), lambda i: (0, 0)),
            pl.BlockSpec((D, G), lambda i: (0, 0)),
            pl.BlockSpec((1, G), lambda i: (0, 0)),
        ],
        out_specs=(
            pl.BlockSpec((TB, G), lambda i: (i, 0)),
            pl.BlockSpec((TB, GO), lambda i: (i, 0)),
            pl.BlockSpec((TB, GO), lambda i: (i, 0)),
        ),
        compiler_params=pltpu.CompilerParams(
            dimension_semantics=("parallel",),
            vmem_limit_bytes=64 * 1024 * 1024,
        ),
    )(x, w_sm, b_sm, w_pi16, b_pi)

    if B_pad != B:
        pi_pad = pi_pad[:B]
        sigma_pad = sigma_pad[:B]
        mu_pad = mu_pad[:B]
    return pi_pad, sigma_pad.reshape(B, G, O), mu_pad.reshape(B, G, O)
